# baseline (speedup 1.0000x reference)
import functools

import jax
import jax.numpy as jnp
from jax import lax
from jax.experimental import pallas as pl
from jax.experimental.pallas import tpu as pltpu
from jax.experimental.pallas import tpu_sc as plsc

B = 16384
D_MLP = 32
D_MF = 16
NC = 2
NS = 16
NW = NC * NS
BPW = B // NW
CH = 128
NCH = BPW // CH

_MESH = plsc.VectorSubcoreMesh(core_axis_name="c", subcore_axis_name="s")


def _sc_gather_body(u_hbm, i_hbm, tum_hbm, tim_hbm, tuf_hbm, tif_hbm,
                    out_um, out_im, out_uf, out_if,
                    idx_u, idx_i, r_um, r_im, r_uf, r_if, sem):
    wid = lax.axis_index("s") * NC + lax.axis_index("c")
    pltpu.sync_copy(u_hbm.at[wid], idx_u)
    pltpu.sync_copy(i_hbm.at[wid], idx_i)
    copies = []
    for j in range(NCH):
        copies.append(pltpu.async_copy(tum_hbm.at[idx_u.at[j]], r_um.at[j], sem))
        copies.append(pltpu.async_copy(tim_hbm.at[idx_i.at[j]], r_im.at[j], sem))
        copies.append(pltpu.async_copy(tuf_hbm.at[idx_u.at[j]], r_uf.at[j], sem))
        copies.append(pltpu.async_copy(tif_hbm.at[idx_i.at[j]], r_if.at[j], sem))
    for c in copies:
        c.wait()
    pltpu.sync_copy(r_um, out_um.at[wid])
    pltpu.sync_copy(r_im, out_im.at[wid])
    pltpu.sync_copy(r_uf, out_uf.at[wid])
    pltpu.sync_copy(r_if, out_if.at[wid])


_sc_gather = functools.partial(
    pl.kernel,
    out_type=[
        jax.ShapeDtypeStruct((NW, NCH, CH, D_MLP), jnp.float32),
        jax.ShapeDtypeStruct((NW, NCH, CH, D_MLP), jnp.float32),
        jax.ShapeDtypeStruct((NW, NCH, CH, D_MF), jnp.float32),
        jax.ShapeDtypeStruct((NW, NCH, CH, D_MF), jnp.float32),
    ],
    mesh=_MESH,
    compiler_params=pltpu.CompilerParams(use_tc_tiling_on_sc=False),
    scratch_types=[
        pltpu.VMEM((NCH, CH), jnp.int32),
        pltpu.VMEM((NCH, CH), jnp.int32),
        pltpu.VMEM((NCH, CH, D_MLP), jnp.float32),
        pltpu.VMEM((NCH, CH, D_MLP), jnp.float32),
        pltpu.VMEM((NCH, CH, D_MF), jnp.float32),
        pltpu.VMEM((NCH, CH, D_MF), jnp.float32),
        pltpu.SemaphoreType.DMA,
    ],
)(_sc_gather_body)


BLK = 1024


def _mlp_body(ug_ref, ig_ref, uf_ref, if_ref, w1u_ref, w1i_ref, b1_ref,
              w2_ref, b2_ref, w3_ref, b3_ref, wo3_ref, womf_ref, bo_ref,
              out_ref):
    f32 = jnp.float32
    h = jnp.dot(ug_ref[...], w1u_ref[...], preferred_element_type=f32)
    h = h + jnp.dot(ig_ref[...], w1i_ref[...], preferred_element_type=f32)
    h = jnp.maximum(h + b1_ref[...], 0.0)
    h = jnp.maximum(jnp.dot(h, w2_ref[...], preferred_element_type=f32) + b2_ref[...], 0.0)
    h = jnp.maximum(jnp.dot(h, w3_ref[...], preferred_element_type=f32) + b3_ref[...], 0.0)
    mf = uf_ref[...] * if_ref[...]
    logits = (jnp.dot(h, wo3_ref[...], preferred_element_type=f32)
              + jnp.dot(mf, womf_ref[...], preferred_element_type=f32)
              + bo_ref[...])
    out_ref[...] = jax.nn.sigmoid(logits)


def _full(shape):
    return pl.BlockSpec(shape, lambda b: (0,) * len(shape))


_mlp = pl.pallas_call(
    _mlp_body,
    grid=(B // BLK,),
    in_specs=[
        pl.BlockSpec((BLK, D_MLP), lambda b: (b, 0)),
        pl.BlockSpec((BLK, D_MLP), lambda b: (b, 0)),
        pl.BlockSpec((BLK, D_MF), lambda b: (b, 0)),
        pl.BlockSpec((BLK, D_MF), lambda b: (b, 0)),
        _full((D_MLP, 32)),
        _full((D_MLP, 32)),
        _full((1, 32)),
        _full((32, 16)),
        _full((1, 16)),
        _full((16, 8)),
        _full((1, 8)),
        _full((8, 1)),
        _full((D_MF, 1)),
        _full((1, 1)),
    ],
    out_specs=pl.BlockSpec((BLK, 1), lambda b: (b, 0)),
    out_shape=jax.ShapeDtypeStruct((B, 1), jnp.float32),
)


def kernel(u, i, emb_user_mlp, emb_item_mlp, emb_user_mf, emb_item_mf,
           W1, b1, W2, b2, W3, b3, W_out, b_out):
    u3 = u.astype(jnp.int32).reshape(NW, NCH, CH)
    i3 = i.astype(jnp.int32).reshape(NW, NCH, CH)
    g_um, g_im, g_uf, g_if = _sc_gather(
        u3, i3, emb_user_mlp, emb_item_mlp, emb_user_mf, emb_item_mf)
    g_um = g_um.reshape(B, D_MLP)
    g_im = g_im.reshape(B, D_MLP)
    g_uf = g_uf.reshape(B, D_MF)
    g_if = g_if.reshape(B, D_MF)
    y = _mlp(g_um, g_im, g_uf, g_if,
             W1[:D_MLP], W1[D_MLP:], b1.reshape(1, -1),
             W2, b2.reshape(1, -1), W3, b3.reshape(1, -1),
             W_out[:8], W_out[8:], b_out.reshape(1, 1))
    return y


# speedup vs baseline: 3.6822x; 3.6822x over previous
import functools

import jax
import jax.numpy as jnp
from jax import lax
from jax.experimental import pallas as pl
from jax.experimental.pallas import tpu as pltpu
from jax.experimental.pallas import tpu_sc as plsc

B = 16384
V = 1000000
D_MLP = 32
D_MF = 16
NC = 2
NS = 16
NW = NC * NS
BPW = B // NW
CROWS = 4
NCHUNK = BPW // CROWS
TAIL0 = (V // 128 - 1) * 128
TAIL_V = V - (TAIL0 + 128)
LANES = 16

_MESH = plsc.VectorSubcoreMesh(core_axis_name="c", subcore_axis_name="s")


def _extract_col(block, panel, col, r):
    nf = block.shape[0]
    colv = jnp.full((LANES,), col, jnp.int32)
    rv = jnp.full((LANES,), r, jnp.int32)
    for f0 in range(0, nf, LANES):
        fidx = lax.iota(jnp.int32, LANES) + f0
        vals = plsc.load_gather(block, [fidx, colv])
        plsc.store_scatter(panel, [fidx, rv], vals)


def _sc_gather_body(u_hbm, i_hbm, tum_hbm, tim_hbm, tuf_hbm, tif_hbm,
                    o_um, o_im, o_uf, o_if,
                    idx_u_v, idx_i_v,
                    b_um, b_im, b_uf, b_if,
                    p_um, p_im, p_uf, p_if, sem):
    wid = lax.axis_index("s") * NC + lax.axis_index("c")
    base = pl.multiple_of(wid * BPW, BPW)
    pltpu.sync_copy(u_hbm.at[wid], idx_u_v)
    pltpu.sync_copy(i_hbm.at[wid], idx_i_v)
    lanes = lax.iota(jnp.int32, LANES)

    def chunk_body(c, carry):
        uvec = idx_u_v.at[c][...]
        ivec = idx_i_v.at[c][...]
        for g in range(LANES // CROWS):
            copies = []
            offs = []
            for k in range(CROWS):
                lane = g * CROWS + k
                uu = jnp.sum(jnp.where(lanes == lane, uvec, 0))
                ii = jnp.sum(jnp.where(lanes == lane, ivec, 0))
                bu = pl.multiple_of(
                    jnp.minimum((uu >> 7) * 128, TAIL0), 128)
                bi = pl.multiple_of(
                    jnp.minimum((ii >> 7) * 128, TAIL0), 128)
                offs.append((jnp.minimum(uu - bu, 127),
                             jnp.minimum(ii - bi, 127)))
                copies.append(pltpu.async_copy(
                    tum_hbm.at[:, pl.ds(bu, 128)], b_um.at[k], sem))
                copies.append(pltpu.async_copy(
                    tim_hbm.at[:, pl.ds(bi, 128)], b_im.at[k], sem))
                copies.append(pltpu.async_copy(
                    tuf_hbm.at[:, pl.ds(bu, 128)], b_uf.at[k], sem))
                copies.append(pltpu.async_copy(
                    tif_hbm.at[:, pl.ds(bi, 128)], b_if.at[k], sem))
            for cp in copies:
                cp.wait()
            for k in range(CROWS):
                r = c * LANES + g * CROWS + k
                cu, ci = offs[k]
                _extract_col(b_um.at[k], p_um, cu, r)
                _extract_col(b_im.at[k], p_im, ci, r)
                _extract_col(b_uf.at[k], p_uf, cu, r)
                _extract_col(b_if.at[k], p_if, ci, r)
        return carry

    lax.fori_loop(0, BPW // LANES, chunk_body, 0)

    pltpu.sync_copy(p_um, o_um.at[:, pl.ds(base, BPW)])
    pltpu.sync_copy(p_im, o_im.at[:, pl.ds(base, BPW)])
    pltpu.sync_copy(p_uf, o_uf.at[:, pl.ds(base, BPW)])
    pltpu.sync_copy(p_if, o_if.at[:, pl.ds(base, BPW)])


_sc_gather = functools.partial(
    pl.kernel,
    out_type=[
        jax.ShapeDtypeStruct((D_MLP, B), jnp.float32),
        jax.ShapeDtypeStruct((D_MLP, B), jnp.float32),
        jax.ShapeDtypeStruct((D_MF, B), jnp.float32),
        jax.ShapeDtypeStruct((D_MF, B), jnp.float32),
    ],
    mesh=_MESH,
    compiler_params=pltpu.CompilerParams(needs_layout_passes=False),
    scratch_types=[
        pltpu.VMEM((BPW // LANES, LANES), jnp.int32),
        pltpu.VMEM((BPW // LANES, LANES), jnp.int32),
        pltpu.VMEM((CROWS, D_MLP, 128), jnp.float32),
        pltpu.VMEM((CROWS, D_MLP, 128), jnp.float32),
        pltpu.VMEM((CROWS, D_MF, 128), jnp.float32),
        pltpu.VMEM((CROWS, D_MF, 128), jnp.float32),
        pltpu.VMEM((D_MLP, BPW), jnp.float32),
        pltpu.VMEM((D_MLP, BPW), jnp.float32),
        pltpu.VMEM((D_MF, BPW), jnp.float32),
        pltpu.VMEM((D_MF, BPW), jnp.float32),
        pltpu.SemaphoreType.DMA,
    ],
)(_sc_gather_body)


BLK = 1024


def _mlp_body(xu_ref, xi_ref, fu_ref, fi_ref, ub_ref, ib_ref,
              tum_ref, tim_ref, tuf_ref, tif_ref,
              w1u_ref, w1i_ref, b1_ref, w2_ref, b2_ref, w3_ref, b3_ref,
              wo3_ref, womf_ref, bo_ref, out_ref):
    f32 = jnp.float32
    ub = ub_ref[...]
    ib = ib_ref[...]
    iot = lax.broadcasted_iota(jnp.int32, (TAIL_V, BLK), 0) + (V - TAIL_V)
    oh_u = (iot == ub).astype(f32)
    oh_i = (iot == ib).astype(f32)
    u_tail = ub >= (V - TAIL_V)
    i_tail = ib >= (V - TAIL_V)
    xu = jnp.where(u_tail, jnp.dot(tum_ref[...], oh_u, preferred_element_type=f32),
                   xu_ref[...])
    xi = jnp.where(i_tail, jnp.dot(tim_ref[...], oh_i, preferred_element_type=f32),
                   xi_ref[...])
    fu = jnp.where(u_tail, jnp.dot(tuf_ref[...], oh_u, preferred_element_type=f32),
                   fu_ref[...])
    fi = jnp.where(i_tail, jnp.dot(tif_ref[...], oh_i, preferred_element_type=f32),
                   fi_ref[...])
    h = jnp.dot(w1u_ref[...], xu, preferred_element_type=f32)
    h = h + jnp.dot(w1i_ref[...], xi, preferred_element_type=f32)
    h = jnp.maximum(h + b1_ref[...], 0.0)
    h = jnp.maximum(jnp.dot(w2_ref[...], h, preferred_element_type=f32) + b2_ref[...], 0.0)
    h = jnp.maximum(jnp.dot(w3_ref[...], h, preferred_element_type=f32) + b3_ref[...], 0.0)
    mf = fu * fi
    logits = (jnp.dot(wo3_ref[...], h, preferred_element_type=f32)
              + jnp.dot(womf_ref[...], mf, preferred_element_type=f32)
              + bo_ref[...])
    out_ref[...] = jax.nn.sigmoid(logits)


def _full(shape):
    return pl.BlockSpec(shape, lambda b: (0,) * len(shape))


_mlp = pl.pallas_call(
    _mlp_body,
    grid=(B // BLK,),
    in_specs=[
        pl.BlockSpec((D_MLP, BLK), lambda b: (0, b)),
        pl.BlockSpec((D_MLP, BLK), lambda b: (0, b)),
        pl.BlockSpec((D_MF, BLK), lambda b: (0, b)),
        pl.BlockSpec((D_MF, BLK), lambda b: (0, b)),
        pl.BlockSpec((1, BLK), lambda b: (0, b)),
        pl.BlockSpec((1, BLK), lambda b: (0, b)),
        _full((D_MLP, TAIL_V)),
        _full((D_MLP, TAIL_V)),
        _full((D_MF, TAIL_V)),
        _full((D_MF, TAIL_V)),
        _full((32, D_MLP)),
        _full((32, D_MLP)),
        _full((32, 1)),
        _full((16, 32)),
        _full((16, 1)),
        _full((8, 16)),
        _full((8, 1)),
        _full((1, 8)),
        _full((1, D_MF)),
        _full((1, 1)),
    ],
    out_specs=pl.BlockSpec((1, BLK), lambda b: (0, b)),
    out_shape=jax.ShapeDtypeStruct((1, B), jnp.float32),
)


def kernel(u, i, emb_user_mlp, emb_item_mlp, emb_user_mf, emb_item_mf,
           W1, b1, W2, b2, W3, b3, W_out, b_out):
    u = u.astype(jnp.int32)
    i = i.astype(jnp.int32)
    g_um, g_im, g_uf, g_if = _sc_gather(
        u.reshape(NW, BPW // LANES, LANES), i.reshape(NW, BPW // LANES, LANES),
        emb_user_mlp.T, emb_item_mlp.T, emb_user_mf.T, emb_item_mf.T)
    y = _mlp(g_um, g_im, g_uf, g_if,
             u.reshape(1, B), i.reshape(1, B),
             emb_user_mlp[V - TAIL_V:].T, emb_item_mlp[V - TAIL_V:].T,
             emb_user_mf[V - TAIL_V:].T, emb_item_mf[V - TAIL_V:].T,
             W1[:D_MLP].T, W1[D_MLP:].T, b1.reshape(-1, 1),
             W2.T, b2.reshape(-1, 1), W3.T, b3.reshape(-1, 1),
             W_out[:8].T, W_out[8:].T, b_out.reshape(1, 1))
    return y.reshape(B, 1)
